# baseline (speedup 1.0000x reference)
import functools
import jax
import jax.numpy as jnp
from jax import lax
from jax.experimental import pallas as pl
from jax.experimental.pallas import tpu as pltpu
from jax.experimental.pallas import tpu_sc as plsc

NC = 2
NS = 16
NW = NC * NS
LANES = 16


def _make_kernel(S, B, D):
    rows = S * B
    rows_per_w = rows // NW
    CH_S = 16
    ch_rows = CH_S * B
    n_chunks = rows_per_w // ch_rows
    vecs = D // LANES

    mesh = plsc.VectorSubcoreMesh(core_axis_name="c", subcore_axis_name="s")

    @functools.partial(
        pl.kernel,
        mesh=mesh,
        out_type=jax.ShapeDtypeStruct((rows, D), jnp.float32),
        scratch_types=[
            pltpu.VMEM((CH_S,), jnp.int32),
            pltpu.VMEM((CH_S, D), jnp.float32),
            pltpu.VMEM((ch_rows, D), jnp.float32),
            pltpu.SemaphoreType.DMA,
            pltpu.SemaphoreType.DMA,
        ],
    )
    def k(x_hbm, idx_hbm, w_hbm, out_hbm, idx_v, w_v, x_v, sem_w, sem_x):
        wid = lax.axis_index("s") * NC + lax.axis_index("c")
        base_s = wid * (rows_per_w // B)

        def chunk_body(c, carry):
            s0 = base_s + c * CH_S
            r0 = s0 * B
            pltpu.sync_copy(idx_hbm.at[pl.ds(s0, CH_S)], idx_v)
            gw = pltpu.async_copy(w_hbm.at[idx_v], w_v, sem_w)
            gx = pltpu.async_copy(x_hbm.at[pl.ds(r0, ch_rows)], x_v, sem_x)
            gw.wait()
            gx.wait()

            def s_body(si, carry2):
                def j_body(j, carry3):
                    wvec = w_v[si, pl.ds(j * LANES, LANES)]
                    for b in range(B):
                        plsc.addupdate(
                            x_v.at[si * B + b, pl.ds(j * LANES, LANES)], wvec)
                    return carry3
                return lax.fori_loop(0, vecs, j_body, carry2)

            carry = lax.fori_loop(0, CH_S, s_body, carry)
            pltpu.sync_copy(x_v, out_hbm.at[pl.ds(r0, ch_rows)])
            return carry

        lax.fori_loop(0, n_chunks, chunk_body, 0)

    return k


@jax.jit
def kernel(x, pos_embed_weight, position):
    S, B, D = x.shape
    idx = position[:S, 0].astype(jnp.int32)
    x2 = x.reshape(S * B, D)
    out = _make_kernel(S, B, D)(x2, idx, pos_embed_weight)
    return out.reshape(S, B, D)


# speedup vs baseline: 1.1604x; 1.1604x over previous
import functools
import jax
import jax.numpy as jnp
from jax import lax
from jax.experimental import pallas as pl
from jax.experimental.pallas import tpu as pltpu
from jax.experimental.pallas import tpu_sc as plsc

NC = 2
NS = 16
NW = NC * NS
LANES = 16


def _make_kernel(S, B, D):
    rows = S * B
    rows_per_w = rows // NW
    s_per_w = rows_per_w // B
    CH_S = 8
    ch_rows = CH_S * B
    n_chunks = s_per_w // CH_S
    vecs = D // LANES

    mesh = plsc.VectorSubcoreMesh(core_axis_name="c", subcore_axis_name="s")

    @functools.partial(
        pl.kernel,
        mesh=mesh,
        out_type=jax.ShapeDtypeStruct((rows, D), jnp.float32),
        scratch_types=[
            pltpu.VMEM((s_per_w,), jnp.int32),
            pltpu.VMEM((CH_S, D), jnp.float32),
            pltpu.VMEM((CH_S, D), jnp.float32),
            pltpu.VMEM((ch_rows, D), jnp.float32),
            pltpu.VMEM((ch_rows, D), jnp.float32),
            pltpu.SemaphoreType.DMA,
            pltpu.SemaphoreType.DMA,
            pltpu.SemaphoreType.DMA,
            pltpu.SemaphoreType.DMA,
            pltpu.SemaphoreType.DMA,
            pltpu.SemaphoreType.DMA,
        ],
    )
    def k(x_hbm, idx_hbm, w_hbm, out_hbm, idx_v, w0, w1, x0, x1,
          sw0, sw1, sx0, sx1, so0, so1):
        wid = lax.axis_index("s") * NC + lax.axis_index("c")
        base_s = wid * s_per_w
        wb = [w0, w1]
        xb = [x0, x1]
        semw = [sw0, sw1]
        semx = [sx0, sx1]
        semo = [so0, so1]

        pltpu.sync_copy(idx_hbm.at[pl.ds(base_s, s_per_w)], idx_v)

        def start_gathers(c, p):
            s0 = base_s + c * CH_S
            gw = pltpu.async_copy(
                w_hbm.at[idx_v.at[pl.ds(c * CH_S, CH_S)]], wb[p], semw[p])
            gx = pltpu.async_copy(
                x_hbm.at[pl.ds(s0 * B, ch_rows)], xb[p], semx[p])
            return gw, gx

        def compute(p):
            x_v, w_v = xb[p], wb[p]

            @plsc.parallel_loop(0, CH_S * vecs, 1, unroll=8)
            def _(i):
                si = i // vecs
                col = (i - si * vecs) * LANES
                wvec = w_v[si, pl.ds(col, LANES)]
                r = si * B
                for b in range(B):
                    plsc.addupdate(x_v.at[r + b, pl.ds(col, LANES)], wvec)

        pend = {}
        out_pend = {}
        pend[0] = start_gathers(0, 0)
        for c in range(n_chunks):
            p = c & 1
            if c + 1 < n_chunks:
                q = 1 - p
                if c >= 1:
                    out_pend[q].wait()
                pend[c + 1] = start_gathers(c + 1, q)
            gw, gx = pend[c]
            gw.wait()
            gx.wait()
            compute(p)
            s0 = base_s + c * CH_S
            out_pend[p] = pltpu.async_copy(
                xb[p], out_hbm.at[pl.ds(s0 * B, ch_rows)], semo[p])
        out_pend[0].wait()
        out_pend[1].wait()

    return k


@jax.jit
def kernel(x, pos_embed_weight, position):
    S, B, D = x.shape
    idx = position[:S, 0].astype(jnp.int32)
    x2 = x.reshape(S * B, D)
    out = _make_kernel(S, B, D)(x2, idx, pos_embed_weight)
    return out.reshape(S, B, D)


# speedup vs baseline: 2.7982x; 2.4114x over previous
import functools
import jax
import jax.numpy as jnp
from jax import lax
from jax.experimental import pallas as pl
from jax.experimental.pallas import tpu as pltpu
from jax.experimental.pallas import tpu_sc as plsc

NC = 2
NS = 16
NW = NC * NS
LANES = 16


def _make_kernel(S, B, D):
    s_per_w = S // NW
    CH_S = 8
    n_chunks = s_per_w // CH_S
    vecs = D // LANES

    mesh = plsc.VectorSubcoreMesh(core_axis_name="c", subcore_axis_name="s")

    @functools.partial(
        pl.kernel,
        mesh=mesh,
        out_type=jax.ShapeDtypeStruct((S, B, D), jnp.float32),
        scratch_types=[
            pltpu.VMEM((s_per_w,), jnp.int32),
            pltpu.VMEM((CH_S, D), jnp.float32),
            pltpu.VMEM((CH_S, D), jnp.float32),
            pltpu.VMEM((CH_S, B, D), jnp.float32),
            pltpu.VMEM((CH_S, B, D), jnp.float32),
            pltpu.SemaphoreType.DMA,
            pltpu.SemaphoreType.DMA,
            pltpu.SemaphoreType.DMA,
            pltpu.SemaphoreType.DMA,
            pltpu.SemaphoreType.DMA,
            pltpu.SemaphoreType.DMA,
        ],
    )
    def k(x_hbm, idx_hbm, w_hbm, out_hbm, idx_v, w0, w1, x0, x1,
          sw0, sw1, sx0, sx1, so0, so1):
        wid = lax.axis_index("s") * NC + lax.axis_index("c")
        base_s = wid * s_per_w
        wb = [w0, w1]
        xb = [x0, x1]
        semw = [sw0, sw1]
        semx = [sx0, sx1]
        semo = [so0, so1]

        pltpu.sync_copy(idx_hbm.at[pl.ds(base_s, s_per_w)], idx_v)

        def start_gathers(c, p):
            s0 = base_s + c * CH_S
            gw = pltpu.async_copy(
                w_hbm.at[idx_v.at[pl.ds(c * CH_S, CH_S)]], wb[p], semw[p])
            gx = pltpu.async_copy(
                x_hbm.at[pl.ds(s0, CH_S)], xb[p], semx[p])
            return gw, gx

        def compute(p):
            x_v, w_v = xb[p], wb[p]

            @plsc.parallel_loop(0, CH_S * vecs, 1, unroll=8)
            def _(i):
                si = i // vecs
                col = (i - si * vecs) * LANES
                wvec = w_v[si, pl.ds(col, LANES)]
                for b in range(B):
                    plsc.addupdate(x_v.at[si, b, pl.ds(col, LANES)], wvec)

        pend = {}
        out_pend = {}
        pend[0] = start_gathers(0, 0)
        for c in range(n_chunks):
            p = c & 1
            if c + 1 < n_chunks:
                q = 1 - p
                if c >= 1:
                    out_pend[q].wait()
                pend[c + 1] = start_gathers(c + 1, q)
            gw, gx = pend[c]
            gw.wait()
            gx.wait()
            compute(p)
            s0 = base_s + c * CH_S
            out_pend[p] = pltpu.async_copy(
                xb[p], out_hbm.at[pl.ds(s0, CH_S)], semo[p])
        out_pend[0].wait()
        out_pend[1].wait()

    return k


@jax.jit
def kernel(x, pos_embed_weight, position):
    S, B, D = x.shape
    idx = position[:S, 0]
    return _make_kernel(S, B, D)(x, idx, pos_embed_weight)


# speedup vs baseline: 2.8374x; 1.0140x over previous
import functools
import jax
import jax.numpy as jnp
from jax import lax
from jax.experimental import pallas as pl
from jax.experimental.pallas import tpu as pltpu
from jax.experimental.pallas import tpu_sc as plsc

NC = 2
NS = 16
NW = NC * NS
LANES = 16


def _make_kernel(S, B, D):
    s_per_w = S // NW
    CH_S = 8
    n_chunks = s_per_w // CH_S
    vecs = D // LANES

    mesh = plsc.VectorSubcoreMesh(core_axis_name="c", subcore_axis_name="s")

    @functools.partial(
        pl.kernel,
        mesh=mesh,
        out_type=jax.ShapeDtypeStruct((S, B, D), jnp.float32),
        scratch_types=[
            pltpu.VMEM((s_per_w,), jnp.int32),
            pltpu.VMEM((CH_S, D), jnp.float32),
            pltpu.VMEM((CH_S, D), jnp.float32),
            pltpu.VMEM((CH_S, D), jnp.float32),
            pltpu.VMEM((CH_S, B, D), jnp.float32),
            pltpu.VMEM((CH_S, B, D), jnp.float32),
            pltpu.VMEM((CH_S, B, D), jnp.float32),
            pltpu.SemaphoreType.DMA,
            pltpu.SemaphoreType.DMA,
            pltpu.SemaphoreType.DMA,
            pltpu.SemaphoreType.DMA,
            pltpu.SemaphoreType.DMA,
            pltpu.SemaphoreType.DMA,
            pltpu.SemaphoreType.DMA,
            pltpu.SemaphoreType.DMA,
            pltpu.SemaphoreType.DMA,
        ],
    )
    def k(x_hbm, idx_hbm, w_hbm, out_hbm, idx_v, w0, w1, w2, x0, x1, x2,
          sw0, sw1, sw2, sx0, sx1, sx2, so0, so1, so2):
        wid = lax.axis_index("s") * NC + lax.axis_index("c")
        base_s = wid * s_per_w
        wb = [w0, w1, w2]
        xb = [x0, x1, x2]
        semw = [sw0, sw1, sw2]
        semx = [sx0, sx1, sx2]
        semo = [so0, so1, so2]

        pltpu.sync_copy(idx_hbm.at[pl.ds(base_s, s_per_w)], idx_v)

        def start_gathers(c, p):
            s0 = base_s + c * CH_S
            gw = pltpu.async_copy(
                w_hbm.at[idx_v.at[pl.ds(c * CH_S, CH_S)]], wb[p], semw[p])
            gx = pltpu.async_copy(
                x_hbm.at[pl.ds(s0, CH_S)], xb[p], semx[p])
            return gw, gx

        def compute(p):
            x_v, w_v = xb[p], wb[p]

            @plsc.parallel_loop(0, CH_S * vecs, 1, unroll=8)
            def _(i):
                si = i // vecs
                col = (i - si * vecs) * LANES
                wvec = w_v[si, pl.ds(col, LANES)]
                for b in range(B):
                    plsc.addupdate(x_v.at[si, b, pl.ds(col, LANES)], wvec)

        pend = {}
        out_pend = {}
        pend[0] = start_gathers(0, 0)
        for c in range(n_chunks):
            p = c % 3
            if c + 1 < n_chunks:
                if c - 2 >= 0:
                    out_pend[c - 2].wait()
                pend[c + 1] = start_gathers(c + 1, (c + 1) % 3)
            gw, gx = pend[c]
            gw.wait()
            gx.wait()
            compute(p)
            s0 = base_s + c * CH_S
            out_pend[c] = pltpu.async_copy(
                xb[p], out_hbm.at[pl.ds(s0, CH_S)], semo[p])
        out_pend[n_chunks - 3].wait()
        out_pend[n_chunks - 2].wait()
        out_pend[n_chunks - 1].wait()

    return k


@jax.jit
def kernel(x, pos_embed_weight, position):
    S, B, D = x.shape
    idx = position[:S, 0]
    return _make_kernel(S, B, D)(x, idx, pos_embed_weight)


# speedup vs baseline: 2.8785x; 1.0145x over previous
import functools
import jax
import jax.numpy as jnp
from jax import lax
from jax.experimental import pallas as pl
from jax.experimental.pallas import tpu as pltpu
from jax.experimental.pallas import tpu_sc as plsc

NC = 2
NS = 16
NW = NC * NS
LANES = 16


def _make_kernel(S, B, D):
    s_per_w = S // NW
    CH_S = 8
    n_chunks = s_per_w // CH_S
    vecs = D // LANES

    mesh = plsc.VectorSubcoreMesh(core_axis_name="c", subcore_axis_name="s")

    @functools.partial(
        pl.kernel,
        mesh=mesh,
        out_type=jax.ShapeDtypeStruct((S, B, D), jnp.float32),
        scratch_types=[
            pltpu.VMEM((s_per_w,), jnp.int32),
            pltpu.VMEM((CH_S, D), jnp.float32),
            pltpu.VMEM((CH_S, D), jnp.float32),
            pltpu.VMEM((CH_S, D), jnp.float32),
            pltpu.VMEM((CH_S, B, D), jnp.float32),
            pltpu.VMEM((CH_S, B, D), jnp.float32),
            pltpu.VMEM((CH_S, B, D), jnp.float32),
            pltpu.SemaphoreType.DMA,
            pltpu.SemaphoreType.DMA,
            pltpu.SemaphoreType.DMA,
            pltpu.SemaphoreType.DMA,
            pltpu.SemaphoreType.DMA,
            pltpu.SemaphoreType.DMA,
            pltpu.SemaphoreType.DMA,
            pltpu.SemaphoreType.DMA,
            pltpu.SemaphoreType.DMA,
        ],
    )
    def k(x_hbm, idx_hbm, w_hbm, out_hbm, idx_v, w0, w1, w2, x0, x1, x2,
          sw0, sw1, sw2, sx0, sx1, sx2, so0, so1, so2):
        wid = lax.axis_index("s") * NC + lax.axis_index("c")
        base_s = wid * s_per_w
        wb = [w0, w1, w2]
        xb = [x0, x1, x2]
        semw = [sw0, sw1, sw2]
        semx = [sx0, sx1, sx2]
        semo = [so0, so1, so2]

        def start_gathers(c, p):
            s0 = base_s + c * CH_S
            gw = pltpu.async_copy(
                w_hbm.at[idx_v.at[pl.ds(c * CH_S, CH_S)]], wb[p], semw[p])
            gx = pltpu.async_copy(
                x_hbm.at[pl.ds(s0, CH_S)], xb[p], semx[p])
            return gw, gx

        gx0 = pltpu.async_copy(x_hbm.at[pl.ds(base_s, CH_S)], xb[0], semx[0])
        pltpu.sync_copy(idx_hbm.at[pl.ds(base_s, s_per_w)], idx_v)
        gw0 = pltpu.async_copy(
            w_hbm.at[idx_v.at[pl.ds(0, CH_S)]], wb[0], semw[0])

        def compute(p):
            x_v, w_v = xb[p], wb[p]

            @plsc.parallel_loop(0, CH_S * vecs, 1, unroll=8)
            def _(i):
                si = i // vecs
                col = (i - si * vecs) * LANES
                wvec = w_v[si, pl.ds(col, LANES)]
                for b in range(B):
                    plsc.addupdate(x_v.at[si, b, pl.ds(col, LANES)], wvec)

        pend = {}
        out_pend = {}
        pend[0] = (gw0, gx0)
        for c in range(n_chunks):
            p = c % 3
            if c + 1 < n_chunks:
                if c - 2 >= 0:
                    out_pend[c - 2].wait()
                pend[c + 1] = start_gathers(c + 1, (c + 1) % 3)
            gw, gx = pend[c]
            gw.wait()
            gx.wait()
            compute(p)
            s0 = base_s + c * CH_S
            out_pend[c] = pltpu.async_copy(
                xb[p], out_hbm.at[pl.ds(s0, CH_S)], semo[p])
        out_pend[n_chunks - 3].wait()
        out_pend[n_chunks - 2].wait()
        out_pend[n_chunks - 1].wait()

    return k


@jax.jit
def kernel(x, pos_embed_weight, position):
    S, B, D = x.shape
    idx = position[:S, 0]
    return _make_kernel(S, B, D)(x, idx, pos_embed_weight)
